# Initial kernel scaffold; baseline (speedup 1.0000x reference)
import jax
import jax.numpy as jnp
from jax.experimental import pallas as pl


def kernel(x, edge_index, edge_type, rel_emb, W_neigh, W_self):
    raise NotImplementedError("write your pallas kernel here")


# speedup vs baseline: 3.3098x; 3.3098x over previous
import functools

import jax
import jax.numpy as jnp
from jax import lax
from jax.experimental import pallas as pl
from jax.experimental.pallas import tpu as pltpu
from jax.experimental.pallas import tpu_sc as plsc

_N = 10000
_E = 320000
_H = 128
_R = 460

_N_PAD = 10240
_ROWS_PER_TILE = _N_PAD // 16
_R_PAD = 512
_CHUNK = 128
_NC, _NS = 2, 16
_NW = _NC * _NS
_E_PAD = 327680
_CHUNKS_PER_W = _E_PAD // (_CHUNK * _NW)
_IDX_BLK = 16
_SLOPE = (1.0 / 8.0 + 1.0 / 3.0) / 2.0


def _sc_accumulate(x, rel_pad, zeros_rows, src2d, dst2d, et2d):
  mesh = plsc.VectorSubcoreMesh(core_axis_name="c", subcore_axis_name="s",
                                num_cores=_NC, num_subcores=_NS)

  @functools.partial(
      pl.kernel,
      out_type=(
          jax.ShapeDtypeStruct((2 * _N_PAD, _H), jnp.float32),
          jax.ShapeDtypeStruct((_NW * _N_PAD,), jnp.float32),
      ),
      mesh=mesh,
      compiler_params=pltpu.CompilerParams(needs_layout_passes=False),
      scratch_types=[
          pltpu.VMEM((_IDX_BLK, _CHUNK), jnp.int32),
          pltpu.VMEM((_IDX_BLK, _CHUNK), jnp.int32),
          pltpu.VMEM((_IDX_BLK, _CHUNK), jnp.int32),
          pltpu.VMEM((_CHUNK, _H), jnp.float32),
          pltpu.VMEM((_N_PAD,), jnp.float32),
          pltpu.VMEM_SHARED((_N_PAD, _H), jnp.float32),
          pltpu.VMEM_SHARED((_R_PAD, _H), jnp.float32),
          pltpu.SemaphoreType.DMA,
      ],
  )
  def k(x_hbm, rel_hbm, z_hbm, src_hbm, dst_hbm, et_hbm, outs_hbm, outd_hbm,
        src_v, dst_v, et_v, rows_v, deg_v, acc_sh, rel_sh, sem):
    c = lax.axis_index("c")
    s = lax.axis_index("s")
    w = c * _NS + s
    row_base = s * _ROWS_PER_TILE

    pltpu.sync_copy(z_hbm, acc_sh.at[pl.ds(row_base, _ROWS_PER_TILE)])

    @pl.when(s == 0)
    def _():
      pltpu.sync_copy(rel_hbm, rel_sh)

    z = jnp.zeros((16,), jnp.float32)
    ones16 = jnp.ones((16,), jnp.float32)

    def zdeg(i, carry):
      deg_v[pl.ds(i * 16, 16)] = z
      return carry

    lax.fori_loop(0, _N_PAD // 16, zdeg, 0)
    plsc.subcore_barrier()

    cb = w * _CHUNKS_PER_W
    for b in range(_CHUNKS_PER_W // _IDX_BLK):
      pltpu.sync_copy(src_hbm.at[pl.ds(cb + b * _IDX_BLK, _IDX_BLK)], src_v)
      pltpu.sync_copy(dst_hbm.at[pl.ds(cb + b * _IDX_BLK, _IDX_BLK)], dst_v)
      pltpu.sync_copy(et_hbm.at[pl.ds(cb + b * _IDX_BLK, _IDX_BLK)], et_v)

      def chunk(j, carry):
        pltpu.async_copy(x_hbm.at[src_v.at[j]], rows_v, sem).wait()
        pltpu.sync_copy(rows_v, acc_sh.at[dst_v.at[j]], add=True)
        pltpu.async_copy(rel_sh.at[et_v.at[j]], rows_v, sem).wait()
        pltpu.sync_copy(rows_v, acc_sh.at[dst_v.at[j]], add=True)
        for g in range(_CHUNK // 16):
          d16 = dst_v[j, pl.ds(g * 16, 16)]
          plsc.addupdate_scatter(deg_v, [d16], ones16)
        return carry

      lax.fori_loop(0, _IDX_BLK, chunk, 0)

    plsc.subcore_barrier()

    out_base = c * _N_PAD + row_base
    pltpu.sync_copy(acc_sh.at[pl.ds(row_base, _ROWS_PER_TILE)],
                    outs_hbm.at[pl.ds(out_base, _ROWS_PER_TILE)])
    pltpu.sync_copy(deg_v, outd_hbm.at[pl.ds(w * _N_PAD, _N_PAD)])

  return k(x, rel_pad, zeros_rows, src2d, dst2d, et2d)


_BLK = 512


def _tc_body(sa_ref, sb_ref, deg_ref, x_ref, wn_ref, ws_ref, o_ref):
  ssum = sa_ref[0] + sb_ref[0]
  deg = jnp.sum(deg_ref[...], axis=0)
  r = 1.0 / jnp.maximum(deg, 1.0)
  h = jnp.dot(ssum * r, wn_ref[...], preferred_element_type=jnp.float32)
  h = h + jnp.dot(x_ref[...], ws_ref[...], preferred_element_type=jnp.float32)
  o_ref[...] = jnp.where(h > 0, h, h * _SLOPE)


def _tc_finish(S3, deg3, x, W_neigh, W_self):
  return pl.pallas_call(
      _tc_body,
      grid=(pl.cdiv(_N, _BLK),),
      in_specs=[
          pl.BlockSpec((1, _BLK, _H), lambda i: (0, i, 0)),
          pl.BlockSpec((1, _BLK, _H), lambda i: (1, i, 0)),
          pl.BlockSpec((_NW, _BLK, 1), lambda i: (0, i, 0)),
          pl.BlockSpec((_BLK, _H), lambda i: (i, 0)),
          pl.BlockSpec((_H, _H), lambda i: (0, 0)),
          pl.BlockSpec((_H, _H), lambda i: (0, 0)),
      ],
      out_specs=pl.BlockSpec((_BLK, _H), lambda i: (i, 0)),
      out_shape=jax.ShapeDtypeStruct((_N, _H), jnp.float32),
  )(S3, S3, deg3, x, W_neigh, W_self)


def kernel(x, edge_index, edge_type, rel_emb, W_neigh, W_self):
  src = edge_index[0].astype(jnp.int32)
  dst = edge_index[1].astype(jnp.int32)
  et = edge_type.astype(jnp.int32)
  pad = _E_PAD - _E
  src2d = jnp.concatenate([src, jnp.zeros((pad,), jnp.int32)]).reshape(-1, _CHUNK)
  dst2d = jnp.concatenate([dst, jnp.full((pad,), _N, jnp.int32)]).reshape(-1, _CHUNK)
  et2d = jnp.concatenate([et, jnp.zeros((pad,), jnp.int32)]).reshape(-1, _CHUNK)
  rel_pad = jnp.concatenate(
      [rel_emb, jnp.zeros((_R_PAD - _R, _H), jnp.float32)], axis=0)
  zeros_rows = jnp.zeros((_ROWS_PER_TILE, _H), jnp.float32)
  S, deg = _sc_accumulate(x, rel_pad, zeros_rows, src2d, dst2d, et2d)
  S3 = S.reshape(2, _N_PAD, _H)
  deg3 = deg.reshape(_NW, _N_PAD, 1)
  return _tc_finish(S3, deg3, x, W_neigh, W_self)


# speedup vs baseline: 3.4108x; 1.0305x over previous
import functools

import jax
import jax.numpy as jnp
from jax import lax
from jax.experimental import pallas as pl
from jax.experimental.pallas import tpu as pltpu
from jax.experimental.pallas import tpu_sc as plsc

_N = 10000
_E = 320000
_H = 128
_R = 460

_N_PAD = 10112
_ROWS_PER_TILE = _N_PAD // 16
_R_PAD = 464
_CHUNK = 128
_NC, _NS = 2, 16
_NW = _NC * _NS
_E_PAD = 327680
_CHUNKS_PER_W = _E_PAD // (_CHUNK * _NW)
_IDX_BLK = 8
_SLOPE = (1.0 / 8.0 + 1.0 / 3.0) / 2.0


def _sc_accumulate(x, rel_pad, zeros_rows, src2d, dst2d, et2d):
  mesh = plsc.VectorSubcoreMesh(core_axis_name="c", subcore_axis_name="s",
                                num_cores=_NC, num_subcores=_NS)

  @functools.partial(
      pl.kernel,
      out_type=(
          jax.ShapeDtypeStruct((2 * _N_PAD, _H), jnp.float32),
          jax.ShapeDtypeStruct((_NW * _N_PAD,), jnp.float32),
      ),
      mesh=mesh,
      compiler_params=pltpu.CompilerParams(needs_layout_passes=False),
      scratch_types=[
          pltpu.VMEM((_IDX_BLK, _CHUNK), jnp.int32),
          pltpu.VMEM((_IDX_BLK, _CHUNK), jnp.int32),
          pltpu.VMEM((_IDX_BLK, _CHUNK), jnp.int32),
          pltpu.VMEM((_CHUNK, _H), jnp.float32),
          pltpu.VMEM((_CHUNK, _H), jnp.float32),
          pltpu.VMEM((_N_PAD,), jnp.float32),
          pltpu.VMEM_SHARED((_N_PAD, _H), jnp.float32),
          pltpu.VMEM_SHARED((_R_PAD, _H), jnp.float32),
          pltpu.SemaphoreType.DMA,
          pltpu.SemaphoreType.DMA,
      ],
  )
  def k(x_hbm, rel_hbm, z_hbm, src_hbm, dst_hbm, et_hbm, outs_hbm, outd_hbm,
        src_v, dst_v, et_v, bufx_v, bufr_v, deg_v, acc_sh, rel_sh,
        gsem, ssem):
    c = lax.axis_index("c")
    s = lax.axis_index("s")
    w = c * _NS + s
    row_base = s * _ROWS_PER_TILE

    pltpu.sync_copy(z_hbm, acc_sh.at[pl.ds(row_base, _ROWS_PER_TILE)])

    @pl.when(s == 0)
    def _():
      pltpu.sync_copy(rel_hbm, rel_sh)

    z = jnp.zeros((16,), jnp.float32)
    ones16 = jnp.ones((16,), jnp.float32)

    def zdeg(i, carry):
      deg_v[pl.ds(i * 16, 16)] = z
      return carry

    lax.fori_loop(0, _N_PAD // 16, zdeg, 0)
    plsc.subcore_barrier()

    cb = w * _CHUNKS_PER_W
    for b in range(_CHUNKS_PER_W // _IDX_BLK):
      pltpu.sync_copy(src_hbm.at[pl.ds(cb + b * _IDX_BLK, _IDX_BLK)], src_v)
      pltpu.sync_copy(dst_hbm.at[pl.ds(cb + b * _IDX_BLK, _IDX_BLK)], dst_v)
      pltpu.sync_copy(et_hbm.at[pl.ds(cb + b * _IDX_BLK, _IDX_BLK)], et_v)

      def chunk(j, carry):
        gx = pltpu.async_copy(x_hbm.at[src_v.at[j]], bufx_v, gsem)
        gr = pltpu.async_copy(rel_sh.at[et_v.at[j]], bufr_v, ssem)
        gx.wait()
        gr.wait()
        pltpu.sync_copy(bufx_v, acc_sh.at[dst_v.at[j]], add=True)
        pltpu.sync_copy(bufr_v, acc_sh.at[dst_v.at[j]], add=True)
        for g in range(_CHUNK // 16):
          d16 = dst_v[j, pl.ds(g * 16, 16)]
          plsc.addupdate_scatter(deg_v, [d16], ones16)
        return carry

      lax.fori_loop(0, _IDX_BLK, chunk, 0)

    plsc.subcore_barrier()

    out_base = c * _N_PAD + row_base
    pltpu.sync_copy(acc_sh.at[pl.ds(row_base, _ROWS_PER_TILE)],
                    outs_hbm.at[pl.ds(out_base, _ROWS_PER_TILE)])
    pltpu.sync_copy(deg_v, outd_hbm.at[pl.ds(w * _N_PAD, _N_PAD)])

  return k(x, rel_pad, zeros_rows, src2d, dst2d, et2d)


_BLK = 400


def _tc_body(sa_ref, sb_ref, deg_ref, x_ref, wn_ref, ws_ref, o_ref):
  ssum = sa_ref[0] + sb_ref[0]
  deg = jnp.sum(deg_ref[...], axis=0)
  r = 1.0 / jnp.maximum(deg, 1.0)
  h = jnp.dot(ssum * r, wn_ref[...], preferred_element_type=jnp.float32)
  h = h + jnp.dot(x_ref[...], ws_ref[...], preferred_element_type=jnp.float32)
  o_ref[...] = jnp.where(h > 0, h, h * _SLOPE)


def _tc_finish(S3, deg3, x, W_neigh, W_self):
  return pl.pallas_call(
      _tc_body,
      grid=(_N // _BLK,),
      in_specs=[
          pl.BlockSpec((1, _BLK, _H), lambda i: (0, i, 0)),
          pl.BlockSpec((1, _BLK, _H), lambda i: (1, i, 0)),
          pl.BlockSpec((_NW, _BLK, 1), lambda i: (0, i, 0)),
          pl.BlockSpec((_BLK, _H), lambda i: (i, 0)),
          pl.BlockSpec((_H, _H), lambda i: (0, 0)),
          pl.BlockSpec((_H, _H), lambda i: (0, 0)),
      ],
      out_specs=pl.BlockSpec((_BLK, _H), lambda i: (i, 0)),
      out_shape=jax.ShapeDtypeStruct((_N, _H), jnp.float32),
  )(S3, S3, deg3, x, W_neigh, W_self)


def kernel(x, edge_index, edge_type, rel_emb, W_neigh, W_self):
  src = edge_index[0].astype(jnp.int32)
  dst = edge_index[1].astype(jnp.int32)
  et = edge_type.astype(jnp.int32)
  pad = _E_PAD - _E
  src2d = jnp.concatenate([src, jnp.zeros((pad,), jnp.int32)]).reshape(-1, _CHUNK)
  dst2d = jnp.concatenate([dst, jnp.full((pad,), _N, jnp.int32)]).reshape(-1, _CHUNK)
  et2d = jnp.concatenate([et, jnp.zeros((pad,), jnp.int32)]).reshape(-1, _CHUNK)
  rel_pad = jnp.concatenate(
      [rel_emb, jnp.zeros((_R_PAD - _R, _H), jnp.float32)], axis=0)
  zeros_rows = jnp.zeros((_ROWS_PER_TILE, _H), jnp.float32)
  S, deg = _sc_accumulate(x, rel_pad, zeros_rows, src2d, dst2d, et2d)
  S3 = S.reshape(2, _N_PAD, _H)
  deg3 = deg.reshape(_NW, _N_PAD, 1)
  return _tc_finish(S3, deg3, x, W_neigh, W_self)


# speedup vs baseline: 3.4635x; 1.0155x over previous
import functools

import jax
import jax.numpy as jnp
from jax import lax
from jax.experimental import pallas as pl
from jax.experimental.pallas import tpu as pltpu
from jax.experimental.pallas import tpu_sc as plsc

_N = 10000
_E = 320000
_H = 128
_R = 460

_N_PAD = 10112
_ROWS_PER_TILE = _N_PAD // 16
_R_PAD = 464
_CHUNK = 128
_NC, _NS = 2, 16
_NW = _NC * _NS
_E_PAD = 327680
_CHUNKS_PER_W = _E_PAD // (_CHUNK * _NW)
_IDX_BLK = 8
_SLOPE = (1.0 / 8.0 + 1.0 / 3.0) / 2.0


def _sc_accumulate(x, rel_pad, zeros_rows, src2d, dst2d, et2d):
  mesh = plsc.VectorSubcoreMesh(core_axis_name="c", subcore_axis_name="s",
                                num_cores=_NC, num_subcores=_NS)

  @functools.partial(
      pl.kernel,
      out_type=(
          jax.ShapeDtypeStruct((2 * _N_PAD, _H), jnp.float32),
          jax.ShapeDtypeStruct((_NW * _N_PAD,), jnp.float32),
      ),
      mesh=mesh,
      compiler_params=pltpu.CompilerParams(needs_layout_passes=False),
      scratch_types=[
          pltpu.VMEM((_IDX_BLK, _CHUNK), jnp.int32),
          pltpu.VMEM((_IDX_BLK, _CHUNK), jnp.int32),
          pltpu.VMEM((_IDX_BLK, _CHUNK), jnp.int32),
          pltpu.VMEM((_CHUNK, _H), jnp.float32),
          pltpu.VMEM((_CHUNK, _H), jnp.float32),
          pltpu.VMEM((_N_PAD,), jnp.float32),
          pltpu.VMEM_SHARED((_N_PAD, _H), jnp.float32),
          pltpu.VMEM_SHARED((_R_PAD, _H), jnp.float32),
          pltpu.SemaphoreType.DMA,
          pltpu.SemaphoreType.DMA,
      ],
  )
  def k(x_hbm, rel_hbm, z_hbm, src_hbm, dst_hbm, et_hbm, outs_hbm, outd_hbm,
        src_v, dst_v, et_v, bufx_v, bufr_v, deg_v, acc_sh, rel_sh,
        gsem, ssem):
    c = lax.axis_index("c")
    s = lax.axis_index("s")
    w = c * _NS + s
    row_base = s * _ROWS_PER_TILE

    pltpu.sync_copy(z_hbm, acc_sh.at[pl.ds(row_base, _ROWS_PER_TILE)])

    @pl.when(s == 0)
    def _():
      pltpu.sync_copy(rel_hbm, rel_sh)

    z = jnp.zeros((16,), jnp.float32)
    ones16 = jnp.ones((16,), jnp.float32)

    def zdeg(i, carry):
      deg_v[pl.ds(i * 16, 16)] = z
      return carry

    lax.fori_loop(0, _N_PAD // 16, zdeg, 0)
    plsc.subcore_barrier()

    cb = w * _CHUNKS_PER_W
    for b in range(_CHUNKS_PER_W // _IDX_BLK):
      pltpu.sync_copy(src_hbm.at[pl.ds(cb + b * _IDX_BLK, _IDX_BLK)], src_v)
      pltpu.sync_copy(dst_hbm.at[pl.ds(cb + b * _IDX_BLK, _IDX_BLK)], dst_v)
      pltpu.sync_copy(et_hbm.at[pl.ds(cb + b * _IDX_BLK, _IDX_BLK)], et_v)

      def chunk(j, carry):
        gx = pltpu.async_copy(x_hbm.at[src_v.at[j]], bufx_v, gsem)
        gr = pltpu.async_copy(rel_sh.at[et_v.at[j]], bufr_v, ssem)
        gx.wait()
        gr.wait()

        def addrow(i, carry):
          for g in range(_H // 16):
            lane = pl.ds(g * 16, 16)
            bufx_v[i, lane] = bufx_v[i, lane] + bufr_v[i, lane]
          return carry

        lax.fori_loop(0, _CHUNK, addrow, 0)
        pltpu.sync_copy(bufx_v, acc_sh.at[dst_v.at[j]], add=True)
        for g in range(_CHUNK // 16):
          d16 = dst_v[j, pl.ds(g * 16, 16)]
          plsc.addupdate_scatter(deg_v, [d16], ones16)
        return carry

      lax.fori_loop(0, _IDX_BLK, chunk, 0)

    plsc.subcore_barrier()

    out_base = c * _N_PAD + row_base
    pltpu.sync_copy(acc_sh.at[pl.ds(row_base, _ROWS_PER_TILE)],
                    outs_hbm.at[pl.ds(out_base, _ROWS_PER_TILE)])
    pltpu.sync_copy(deg_v, outd_hbm.at[pl.ds(w * _N_PAD, _N_PAD)])

  return k(x, rel_pad, zeros_rows, src2d, dst2d, et2d)


_BLK = 400


def _tc_body(sa_ref, sb_ref, deg_ref, x_ref, wn_ref, ws_ref, o_ref):
  ssum = sa_ref[0] + sb_ref[0]
  deg = jnp.sum(deg_ref[...], axis=0)
  r = 1.0 / jnp.maximum(deg, 1.0)
  h = jnp.dot(ssum * r, wn_ref[...], preferred_element_type=jnp.float32)
  h = h + jnp.dot(x_ref[...], ws_ref[...], preferred_element_type=jnp.float32)
  o_ref[...] = jnp.where(h > 0, h, h * _SLOPE)


def _tc_finish(S3, deg3, x, W_neigh, W_self):
  return pl.pallas_call(
      _tc_body,
      grid=(_N // _BLK,),
      in_specs=[
          pl.BlockSpec((1, _BLK, _H), lambda i: (0, i, 0)),
          pl.BlockSpec((1, _BLK, _H), lambda i: (1, i, 0)),
          pl.BlockSpec((_NW, _BLK, 1), lambda i: (0, i, 0)),
          pl.BlockSpec((_BLK, _H), lambda i: (i, 0)),
          pl.BlockSpec((_H, _H), lambda i: (0, 0)),
          pl.BlockSpec((_H, _H), lambda i: (0, 0)),
      ],
      out_specs=pl.BlockSpec((_BLK, _H), lambda i: (i, 0)),
      out_shape=jax.ShapeDtypeStruct((_N, _H), jnp.float32),
  )(S3, S3, deg3, x, W_neigh, W_self)


def kernel(x, edge_index, edge_type, rel_emb, W_neigh, W_self):
  src = edge_index[0].astype(jnp.int32)
  dst = edge_index[1].astype(jnp.int32)
  et = edge_type.astype(jnp.int32)
  pad = _E_PAD - _E
  src2d = jnp.concatenate([src, jnp.zeros((pad,), jnp.int32)]).reshape(-1, _CHUNK)
  dst2d = jnp.concatenate([dst, jnp.full((pad,), _N, jnp.int32)]).reshape(-1, _CHUNK)
  et2d = jnp.concatenate([et, jnp.zeros((pad,), jnp.int32)]).reshape(-1, _CHUNK)
  rel_pad = jnp.concatenate(
      [rel_emb, jnp.zeros((_R_PAD - _R, _H), jnp.float32)], axis=0)
  zeros_rows = jnp.zeros((_ROWS_PER_TILE, _H), jnp.float32)
  S, deg = _sc_accumulate(x, rel_pad, zeros_rows, src2d, dst2d, et2d)
  S3 = S.reshape(2, _N_PAD, _H)
  deg3 = deg.reshape(_NW, _N_PAD, 1)
  return _tc_finish(S3, deg3, x, W_neigh, W_self)
